# baseline (speedup 1.0000x reference)
import functools

import jax
import jax.numpy as jnp
from jax.experimental import pallas as pl
from jax.experimental.pallas import tpu as pltpu
from jax.experimental.pallas import tpu_sc as plsc

RATIO = 0.5


def _select_body(s_ref, srow_ref, h_ref, rank_ref, idx_ref, idx2e_ref,
                 newh_ref, *, e):
    n, d = h_ref.shape
    k = idx_ref.shape[0]
    cb = 256

    scol = s_ref[...]
    srow = srow_ref[...]

    for c in range(n // cb):
        si = srow[:, c * cb:(c + 1) * cb]
        ii = c * cb + jax.lax.broadcasted_iota(jnp.int32, (n, cb), 1)
        jj = jax.lax.broadcasted_iota(jnp.int32, (n, cb), 0)
        beats = (scol > si) | ((scol == si) & (jj < ii))
        rank_ref[:, c * cb:(c + 1) * cb] = jnp.sum(
            beats.astype(jnp.int32), axis=0, keepdims=True)

    rank_row = rank_ref[...]

    for c in range(k // cb):
        pp = c * cb + jax.lax.broadcasted_iota(jnp.int32, (cb, n), 0)
        jj = jax.lax.broadcasted_iota(jnp.int32, (cb, n), 1)
        val = jnp.sum(jnp.where(rank_row == pp, jj, 0), axis=1, keepdims=True)
        idx_ref[c * cb:(c + 1) * cb, :] = val

    idx = idx_ref[...]
    idx2e_ref[pl.ds(0, k), :] = idx
    idx2e_ref[pl.ds(k, k), :] = idx + e

    p_f = (rank_row == jax.lax.broadcasted_iota(jnp.int32, (k, n), 0)
           ).astype(jnp.float32)
    vals = jax.lax.dot_general(p_f, scol, (((1,), (0,)), ((), ())),
                               precision=jax.lax.Precision.HIGHEST,
                               preferred_element_type=jnp.float32)
    hk = jax.lax.dot_general(p_f, h_ref[...], (((1,), (0,)), ((), ())),
                             precision=jax.lax.Precision.HIGHEST,
                             preferred_element_type=jnp.float32)
    newh_ref[...] = hk * vals


def _select(s, srow, h, k, e, interpret=False):
    n, d = h.shape
    body = functools.partial(_select_body, e=e)
    return pl.pallas_call(
        body,
        out_shape=(
            jax.ShapeDtypeStruct((1, n), jnp.int32),
            jax.ShapeDtypeStruct((k, 1), jnp.int32),
            jax.ShapeDtypeStruct((2 * k, 1), jnp.int32),
            jax.ShapeDtypeStruct((k, d), jnp.float32),
        ),
        interpret=interpret,
    )(s, srow, h)


def _adj_scatter(edge_flat, n, e):
    nn = n * n
    half = nn // 2
    mesh = plsc.VectorSubcoreMesh(core_axis_name="c", subcore_axis_name="s")
    per_tile = e // 16
    quarter = nn // 4
    zwords = 16384
    zchunks = quarter // 16 // zwords
    n_scat = per_tile // 128
    spad = 4096

    @functools.partial(
        pl.kernel,
        out_type=jax.ShapeDtypeStruct((nn,), jnp.float32),
        mesh=mesh,
        scratch_types=[
            pltpu.VMEM_SHARED((quarter + spad,), jnp.float32),
            pltpu.VMEM((zwords,), jnp.float32),
            pltpu.VMEM((per_tile,), jnp.int32),
            pltpu.VMEM((per_tile,), jnp.int32),
            pltpu.VMEM((per_tile // 128, 128), jnp.int32),
            pltpu.VMEM((128,), jnp.float32),
            pltpu.SemaphoreType.DMA,
            pltpu.SemaphoreType.DMA,
            pltpu.SemaphoreType.DMA,
            pltpu.SemaphoreType.DMA,
        ],
    )
    def adj_kernel(ef_hbm, gb_hbm, stage, zbuf, rbuf, cbuf, flbuf, ones_v,
                   zsem, esem, ssem, wsem):
        cid = jax.lax.axis_index("c")
        sid = jax.lax.axis_index("s")
        wid = sid * 2 + cid

        ecopies = [
            pltpu.async_copy(ef_hbm.at[pl.ds(sid * per_tile, per_tile)],
                             rbuf, esem),
            pltpu.async_copy(ef_hbm.at[pl.ds(e + sid * per_tile, per_tile)],
                             cbuf, esem),
        ]

        @pl.loop(0, zwords, step=16)
        def _(i):
            zbuf[pl.ds(i, 16)] = jnp.zeros((16,), jnp.float32)

        @pl.loop(0, 128, step=16)
        def _(i):
            ones_v[pl.ds(i, 16)] = jnp.full((16,), 1.0, jnp.float32)

        for c in ecopies:
            c.wait()

        tile_words = quarter // 16
        for q in range(2):
            region_lo = cid * half + q * quarter

            zcopies = [
                pltpu.async_copy(
                    zbuf,
                    stage.at[pl.ds(sid * tile_words + z * zwords, zwords)],
                    zsem)
                for z in range(zchunks)
            ]

            @pl.loop(0, per_tile // 16)
            def _(i):
                rv = rbuf[pl.ds(i * 16, 16)]
                cv = cbuf[pl.ds(i * 16, 16)]
                fl = rv * n + cv - region_lo
                mine = (fl >= 0) & (fl < quarter)
                dummy = quarter + ((wid * per_tile + i * 16
                                    + jax.lax.iota(jnp.int32, 16))
                                   & (spad - 1))
                fl2 = jnp.where(mine, fl, dummy)
                flbuf[i // 8, pl.ds((i % 8) * 16, 16)] = fl2

            for c in zcopies:
                c.wait()
            plsc.subcore_barrier()

            scopies = [
                pltpu.async_copy(ones_v, stage.at[flbuf.at[j]], ssem,
                                 add=True)
                for j in range(n_scat)
            ]
            for c in scopies:
                c.wait()
            plsc.subcore_barrier()

            pltpu.async_copy(
                stage.at[pl.ds(sid * tile_words, tile_words)],
                gb_hbm.at[pl.ds(region_lo + sid * tile_words, tile_words)],
                wsem).wait()
            plsc.subcore_barrier()

    return adj_kernel(edge_flat)


def _edge_gather(edge_flat, idx2e, k):
    mesh = plsc.VectorSubcoreMesh(core_axis_name="c", subcore_axis_name="s")
    n_workers = 32
    per_tile = (2 * k) // n_workers

    @functools.partial(
        pl.kernel,
        out_type=jax.ShapeDtypeStruct((2 * k,), jnp.int32),
        mesh=mesh,
        scratch_types=[
            pltpu.VMEM((per_tile,), jnp.int32),
            pltpu.VMEM((per_tile,), jnp.int32),
            pltpu.SemaphoreType.DMA,
        ],
    )
    def gather_kernel(ef_hbm, i_hbm, out_hbm, iv, ov, sem):
        cid = jax.lax.axis_index("c")
        sid = jax.lax.axis_index("s")
        wid = sid * 2 + cid
        base = wid * per_tile
        pltpu.sync_copy(i_hbm.at[pl.ds(base, per_tile)], iv)
        pltpu.async_copy(ef_hbm.at[iv], ov, sem).wait()
        pltpu.sync_copy(ov, out_hbm.at[pl.ds(base, per_tile)])

    return gather_kernel(edge_flat, idx2e)


def _power_body(gb_ref, rrow_ref, rcol_ref, out_ref, gbq, gbsel, *, blk, mmt):
    n = gbq.shape[0]
    k = gbsel.shape[1]
    i = pl.program_id(0)

    @pl.when(i == 0)
    def _():
        gbq[...] = (gb_ref[...].reshape(gbq.shape) != 0).astype(mmt)
        qq = jax.lax.broadcasted_iota(jnp.int32, (n, k), 1)
        ptb = (rcol_ref[...] == qq).astype(mmt)
        sel = jax.lax.dot_general(gbq[...], ptb, (((1,), (0,)), ((), ())),
                                  preferred_element_type=jnp.float32)
        gbsel[...] = (sel != 0).astype(mmt)

    rank_row = rrow_ref[...]
    pp = blk * i + jax.lax.broadcasted_iota(jnp.int32, (blk, n), 0)
    p_blk = (rank_row == pp).astype(mmt)

    l1 = jax.lax.dot_general(p_blk, gbq[...], (((1,), (0,)), ((), ())),
                             preferred_element_type=jnp.float32)
    b1 = (l1 != 0).astype(mmt)
    l2 = jax.lax.dot_general(b1, gbq[...], (((1,), (0,)), ((), ())),
                             preferred_element_type=jnp.float32)
    b2 = (l2 != 0).astype(mmt)
    l3 = jax.lax.dot_general(b2, gbsel[...], (((1,), (0,)), ((), ())),
                             preferred_element_type=jnp.float32)
    un = (l3 != 0).astype(jnp.float32)
    rs = jnp.sum(un, axis=1, keepdims=True)
    out_ref[...] = un / (rs + 1e-8)


def _power_norm(gb_flat, rank_row, rank_col, k, interpret=False,
                mmt=jnp.float8_e4m3fn):
    n = rank_row.shape[1]
    blk = 256
    body = functools.partial(_power_body, blk=blk, mmt=mmt)
    return pl.pallas_call(
        body,
        grid=(k // blk,),
        out_shape=jax.ShapeDtypeStruct((k, k), jnp.float32),
        in_specs=[
            pl.BlockSpec((n * n,), lambda i: (0,)),
            pl.BlockSpec((1, n), lambda i: (0, 0)),
            pl.BlockSpec((n, 1), lambda i: (0, 0)),
        ],
        out_specs=pl.BlockSpec((blk, k), lambda i: (i, 0)),
        scratch_shapes=[
            pltpu.VMEM((n, n), mmt),
            pltpu.VMEM((n, k), mmt),
        ],
        interpret=interpret,
    )(gb_flat, rank_row, rank_col)


def kernel(edge_index, h, C, Wf, bf, Ws, bs, Wo, bo):
    n, d = h.shape
    e = edge_index.shape[1]
    k = max(2, int(RATIO * n))

    feature_weights = h @ Wf + bf
    structure_weights = C @ Ws + bs
    weights = (jnp.concatenate([feature_weights, structure_weights], axis=1)
               @ Wo + bo).squeeze()
    s = jax.nn.sigmoid(weights).reshape(n, 1)

    rank_row, idx, idx2e, new_h = _select(s, s.T, h, k, e)

    edge_flat = edge_index.reshape(2 * e)
    gb_flat = _adj_scatter(edge_flat, n, e)
    new_edge_flat = _edge_gather(edge_flat, idx2e.reshape(2 * k), k)

    g_out = _power_norm(gb_flat, rank_row, rank_row.T, k)

    return (g_out, new_h, idx.reshape(k), new_edge_flat.reshape(2, k))


# speedup vs baseline: 1.1067x; 1.1067x over previous
import functools

import jax
import jax.numpy as jnp
from jax.experimental import pallas as pl
from jax.experimental.pallas import tpu as pltpu
from jax.experimental.pallas import tpu_sc as plsc

RATIO = 0.5


def _select_body(s_ref, srow_ref, h_ref, rank_ref, idx_ref, idx2e_ref,
                 newh_ref, *, e):
    n, d = h_ref.shape
    k = idx_ref.shape[0]
    cb = 256

    scol = s_ref[...]
    srow = srow_ref[...]

    for c in range(n // cb):
        si = srow[:, c * cb:(c + 1) * cb]
        ii = c * cb + jax.lax.broadcasted_iota(jnp.int32, (n, cb), 1)
        jj = jax.lax.broadcasted_iota(jnp.int32, (n, cb), 0)
        beats = (scol > si) | ((scol == si) & (jj < ii))
        rank_ref[:, c * cb:(c + 1) * cb] = jnp.sum(
            beats.astype(jnp.int32), axis=0, keepdims=True)

    rank_row = rank_ref[...]

    for c in range(k // cb):
        pp = c * cb + jax.lax.broadcasted_iota(jnp.int32, (cb, n), 0)
        jj = jax.lax.broadcasted_iota(jnp.int32, (cb, n), 1)
        val = jnp.sum(jnp.where(rank_row == pp, jj, 0), axis=1, keepdims=True)
        idx_ref[c * cb:(c + 1) * cb, :] = val

    idx = idx_ref[...]
    idx2e_ref[pl.ds(0, k), :] = idx
    idx2e_ref[pl.ds(k, k), :] = idx + e

    p_f = (rank_row == jax.lax.broadcasted_iota(jnp.int32, (k, n), 0)
           ).astype(jnp.float32)
    vals = jax.lax.dot_general(p_f, scol, (((1,), (0,)), ((), ())),
                               precision=jax.lax.Precision.HIGHEST,
                               preferred_element_type=jnp.float32)
    hk = jax.lax.dot_general(p_f, h_ref[...], (((1,), (0,)), ((), ())),
                             precision=jax.lax.Precision.HIGHEST,
                             preferred_element_type=jnp.float32)
    newh_ref[...] = hk * vals


def _select(s, srow, h, k, e, interpret=False):
    n, d = h.shape
    body = functools.partial(_select_body, e=e)
    return pl.pallas_call(
        body,
        out_shape=(
            jax.ShapeDtypeStruct((1, n), jnp.int32),
            jax.ShapeDtypeStruct((k, 1), jnp.int32),
            jax.ShapeDtypeStruct((2 * k, 1), jnp.int32),
            jax.ShapeDtypeStruct((k, d), jnp.float32),
        ),
        interpret=interpret,
    )(s, srow, h)


def _adj_scatter(edge_flat, n, e):
    nn = n * n
    half = nn // 2
    mesh = plsc.VectorSubcoreMesh(core_axis_name="c", subcore_axis_name="s")
    per_tile = e // 16
    quarter = nn // 4
    zwords = 16384
    zchunks = quarter // 16 // zwords
    n_scat = per_tile // 128
    spad = 4096

    @functools.partial(
        pl.kernel,
        out_type=jax.ShapeDtypeStruct((nn,), jnp.float32),
        mesh=mesh,
        scratch_types=[
            pltpu.VMEM_SHARED((quarter + spad,), jnp.float32),
            pltpu.VMEM((zwords,), jnp.float32),
            pltpu.VMEM((per_tile,), jnp.int32),
            pltpu.VMEM((per_tile,), jnp.int32),
            pltpu.VMEM((per_tile // 128, 128), jnp.int32),
            pltpu.VMEM((128,), jnp.float32),
            pltpu.SemaphoreType.DMA,
            pltpu.SemaphoreType.DMA,
            pltpu.SemaphoreType.DMA,
            pltpu.SemaphoreType.DMA,
        ],
    )
    def adj_kernel(ef_hbm, gb_hbm, stage, zbuf, rbuf, cbuf, flbuf, ones_v,
                   zsem, esem, ssem, wsem):
        cid = jax.lax.axis_index("c")
        sid = jax.lax.axis_index("s")
        wid = sid * 2 + cid

        ecopies = [
            pltpu.async_copy(ef_hbm.at[pl.ds(sid * per_tile, per_tile)],
                             rbuf, esem),
            pltpu.async_copy(ef_hbm.at[pl.ds(e + sid * per_tile, per_tile)],
                             cbuf, esem),
        ]

        @pl.loop(0, zwords, step=16)
        def _(i):
            zbuf[pl.ds(i, 16)] = jnp.zeros((16,), jnp.float32)

        @pl.loop(0, 128, step=16)
        def _(i):
            ones_v[pl.ds(i, 16)] = jnp.full((16,), 1.0, jnp.float32)

        for c in ecopies:
            c.wait()

        tile_words = quarter // 16
        for q in range(2):
            region_lo = cid * half + q * quarter

            zcopies = [
                pltpu.async_copy(
                    zbuf,
                    stage.at[pl.ds(sid * tile_words + z * zwords, zwords)],
                    zsem)
                for z in range(zchunks)
            ]

            @pl.loop(0, per_tile // 16)
            def _(i):
                rv = rbuf[pl.ds(i * 16, 16)]
                cv = cbuf[pl.ds(i * 16, 16)]
                fl = rv * n + cv - region_lo
                mine = (fl >= 0) & (fl < quarter)
                dummy = quarter + ((wid * per_tile + i * 16
                                    + jax.lax.iota(jnp.int32, 16))
                                   & (spad - 1))
                fl2 = jnp.where(mine, fl, dummy)
                flbuf[i // 8, pl.ds((i % 8) * 16, 16)] = fl2

            for c in zcopies:
                c.wait()
            plsc.subcore_barrier()

            scopies = [
                pltpu.async_copy(ones_v, stage.at[flbuf.at[j]], ssem,
                                 add=True)
                for j in range(n_scat)
            ]
            for c in scopies:
                c.wait()
            plsc.subcore_barrier()

            pltpu.async_copy(
                stage.at[pl.ds(sid * tile_words, tile_words)],
                gb_hbm.at[pl.ds(region_lo + sid * tile_words, tile_words)],
                wsem).wait()
            plsc.subcore_barrier()

    return adj_kernel(edge_flat)


def _edge_gather(edge_flat, idx2e, k):
    mesh = plsc.VectorSubcoreMesh(core_axis_name="c", subcore_axis_name="s")
    n_workers = 32
    per_tile = (2 * k) // n_workers

    @functools.partial(
        pl.kernel,
        out_type=jax.ShapeDtypeStruct((2 * k,), jnp.int32),
        mesh=mesh,
        scratch_types=[
            pltpu.VMEM((per_tile,), jnp.int32),
            pltpu.VMEM((per_tile,), jnp.int32),
            pltpu.SemaphoreType.DMA,
        ],
    )
    def gather_kernel(ef_hbm, i_hbm, out_hbm, iv, ov, sem):
        cid = jax.lax.axis_index("c")
        sid = jax.lax.axis_index("s")
        wid = sid * 2 + cid
        base = wid * per_tile
        pltpu.sync_copy(i_hbm.at[pl.ds(base, per_tile)], iv)
        pltpu.async_copy(ef_hbm.at[iv], ov, sem).wait()
        pltpu.sync_copy(ov, out_hbm.at[pl.ds(base, per_tile)])

    return gather_kernel(edge_flat, idx2e)


def _power_body(gb_ref, rrow_ref, rcol_ref, out_ref, gbq, gbsel, *, blk, mmt):
    n = gbq.shape[0]
    k = gbsel.shape[1]
    i = pl.program_id(0)

    @pl.when(i == 0)
    def _():
        gbq[...] = (gb_ref[...] != 0).astype(mmt)
        qq = jax.lax.broadcasted_iota(jnp.int32, (n, k), 1)
        ptb = (rcol_ref[...] == qq).astype(mmt)
        sel = jax.lax.dot_general(gbq[...], ptb, (((1,), (0,)), ((), ())),
                                  preferred_element_type=jnp.float32)
        gbsel[...] = (sel != 0).astype(mmt)

    rank_row = rrow_ref[...]
    pp = blk * i + jax.lax.broadcasted_iota(jnp.int32, (blk, n), 0)
    p_blk = (rank_row == pp).astype(mmt)

    l1 = jax.lax.dot_general(p_blk, gbq[...], (((1,), (0,)), ((), ())),
                             preferred_element_type=jnp.float32)
    b1 = (l1 != 0).astype(mmt)
    l2 = jax.lax.dot_general(b1, gbq[...], (((1,), (0,)), ((), ())),
                             preferred_element_type=jnp.float32)
    b2 = (l2 != 0).astype(mmt)
    l3 = jax.lax.dot_general(b2, gbsel[...], (((1,), (0,)), ((), ())),
                             preferred_element_type=jnp.float32)
    un = (l3 != 0).astype(jnp.float32)
    rs = jnp.sum(un, axis=1, keepdims=True)
    out_ref[...] = un / (rs + 1e-8)


def _power_norm(gb_flat, rank_row, rank_col, k, interpret=False,
                mmt=jnp.float8_e4m3fn):
    n = rank_row.shape[1]
    blk = 256
    body = functools.partial(_power_body, blk=blk, mmt=mmt)
    return pl.pallas_call(
        body,
        grid=(k // blk,),
        out_shape=jax.ShapeDtypeStruct((k, k), jnp.float32),
        in_specs=[
            pl.BlockSpec((n, n), lambda i: (0, 0)),
            pl.BlockSpec((1, n), lambda i: (0, 0)),
            pl.BlockSpec((n, 1), lambda i: (0, 0)),
        ],
        out_specs=pl.BlockSpec((blk, k), lambda i: (i, 0)),
        scratch_shapes=[
            pltpu.VMEM((n, n), mmt),
            pltpu.VMEM((n, k), mmt),
        ],
        interpret=interpret,
    )(gb_flat, rank_row, rank_col)


def kernel(edge_index, h, C, Wf, bf, Ws, bs, Wo, bo):
    n, d = h.shape
    e = edge_index.shape[1]
    k = max(2, int(RATIO * n))

    feature_weights = h @ Wf + bf
    structure_weights = C @ Ws + bs
    weights = (jnp.concatenate([feature_weights, structure_weights], axis=1)
               @ Wo + bo).squeeze()
    s = jax.nn.sigmoid(weights).reshape(n, 1)

    rank_row, idx, idx2e, new_h = _select(s, s.T, h, k, e)

    edge_flat = edge_index.reshape(2 * e)
    gb_flat = _adj_scatter(edge_flat, n, e)
    new_edge_flat = _edge_gather(edge_flat, idx2e.reshape(2 * k), k)

    g_out = _power_norm(gb_flat.reshape(n, n), rank_row, rank_row.T, k)

    return (g_out, new_h, idx.reshape(k), new_edge_flat.reshape(2, k))


# speedup vs baseline: 1.1074x; 1.0006x over previous
import functools

import jax
import jax.numpy as jnp
from jax.experimental import pallas as pl
from jax.experimental.pallas import tpu as pltpu
from jax.experimental.pallas import tpu_sc as plsc

RATIO = 0.5


def _select_body(s_ref, srow_ref, h_ref, rank_ref, idx_ref, idx2e_ref,
                 newh_ref, *, e):
    n, d = h_ref.shape
    k = idx_ref.shape[0]
    cb = 512

    scol = s_ref[...]
    srow = srow_ref[...]

    for c in range(n // cb):
        si = srow[:, c * cb:(c + 1) * cb]
        ii = c * cb + jax.lax.broadcasted_iota(jnp.int32, (n, cb), 1)
        jj = jax.lax.broadcasted_iota(jnp.int32, (n, cb), 0)
        beats = (scol > si) | ((scol == si) & (jj < ii))
        rank_ref[:, c * cb:(c + 1) * cb] = jnp.sum(
            beats.astype(jnp.int32), axis=0, keepdims=True)

    rank_row = rank_ref[...]

    for c in range(k // cb):
        pp = c * cb + jax.lax.broadcasted_iota(jnp.int32, (cb, n), 0)
        jj = jax.lax.broadcasted_iota(jnp.int32, (cb, n), 1)
        val = jnp.sum(jnp.where(rank_row == pp, jj, 0), axis=1, keepdims=True)
        idx_ref[c * cb:(c + 1) * cb, :] = val

    idx = idx_ref[...]
    idx2e_ref[pl.ds(0, k), :] = idx
    idx2e_ref[pl.ds(k, k), :] = idx + e

    p_f = (rank_row == jax.lax.broadcasted_iota(jnp.int32, (k, n), 0)
           ).astype(jnp.float32)
    vals = jax.lax.dot_general(p_f, scol, (((1,), (0,)), ((), ())),
                               precision=jax.lax.Precision.HIGHEST,
                               preferred_element_type=jnp.float32)
    hk = jax.lax.dot_general(p_f, h_ref[...], (((1,), (0,)), ((), ())),
                             precision=jax.lax.Precision.HIGHEST,
                             preferred_element_type=jnp.float32)
    newh_ref[...] = hk * vals


def _select(s, srow, h, k, e, interpret=False):
    n, d = h.shape
    body = functools.partial(_select_body, e=e)
    return pl.pallas_call(
        body,
        out_shape=(
            jax.ShapeDtypeStruct((1, n), jnp.int32),
            jax.ShapeDtypeStruct((k, 1), jnp.int32),
            jax.ShapeDtypeStruct((2 * k, 1), jnp.int32),
            jax.ShapeDtypeStruct((k, d), jnp.float32),
        ),
        interpret=interpret,
    )(s, srow, h)


def _adj_scatter(edge_flat, n, e):
    nn = n * n
    half = nn // 2
    mesh = plsc.VectorSubcoreMesh(core_axis_name="c", subcore_axis_name="s")
    per_tile = e // 16
    quarter = nn // 4
    zwords = 16384
    zchunks = quarter // 16 // zwords
    n_scat = per_tile // 128
    spad = 4096

    @functools.partial(
        pl.kernel,
        out_type=jax.ShapeDtypeStruct((nn,), jnp.float32),
        mesh=mesh,
        scratch_types=[
            pltpu.VMEM_SHARED((quarter + spad,), jnp.float32),
            pltpu.VMEM((zwords,), jnp.float32),
            pltpu.VMEM((per_tile,), jnp.int32),
            pltpu.VMEM((per_tile,), jnp.int32),
            pltpu.VMEM((per_tile // 128, 128), jnp.int32),
            pltpu.VMEM((128,), jnp.float32),
            pltpu.SemaphoreType.DMA,
            pltpu.SemaphoreType.DMA,
            pltpu.SemaphoreType.DMA,
            pltpu.SemaphoreType.DMA,
        ],
    )
    def adj_kernel(ef_hbm, gb_hbm, stage, zbuf, rbuf, cbuf, flbuf, ones_v,
                   zsem, esem, ssem, wsem):
        cid = jax.lax.axis_index("c")
        sid = jax.lax.axis_index("s")
        wid = sid * 2 + cid

        ecopies = [
            pltpu.async_copy(ef_hbm.at[pl.ds(sid * per_tile, per_tile)],
                             rbuf, esem),
            pltpu.async_copy(ef_hbm.at[pl.ds(e + sid * per_tile, per_tile)],
                             cbuf, esem),
        ]

        @pl.loop(0, zwords, step=16)
        def _(i):
            zbuf[pl.ds(i, 16)] = jnp.zeros((16,), jnp.float32)

        @pl.loop(0, 128, step=16)
        def _(i):
            ones_v[pl.ds(i, 16)] = jnp.full((16,), 1.0, jnp.float32)

        for c in ecopies:
            c.wait()

        tile_words = quarter // 16
        for q in range(2):
            region_lo = cid * half + q * quarter

            zcopies = [
                pltpu.async_copy(
                    zbuf,
                    stage.at[pl.ds(sid * tile_words + z * zwords, zwords)],
                    zsem)
                for z in range(zchunks)
            ]

            @pl.loop(0, per_tile // 16)
            def _(i):
                rv = rbuf[pl.ds(i * 16, 16)]
                cv = cbuf[pl.ds(i * 16, 16)]
                fl = rv * n + cv - region_lo
                mine = (fl >= 0) & (fl < quarter)
                dummy = quarter + ((wid * per_tile + i * 16
                                    + jax.lax.iota(jnp.int32, 16))
                                   & (spad - 1))
                fl2 = jnp.where(mine, fl, dummy)
                flbuf[i // 8, pl.ds((i % 8) * 16, 16)] = fl2

            for c in zcopies:
                c.wait()
            plsc.subcore_barrier()

            scopies = [
                pltpu.async_copy(ones_v, stage.at[flbuf.at[j]], ssem,
                                 add=True)
                for j in range(n_scat)
            ]
            for c in scopies:
                c.wait()
            plsc.subcore_barrier()

            pltpu.async_copy(
                stage.at[pl.ds(sid * tile_words, tile_words)],
                gb_hbm.at[pl.ds(region_lo + sid * tile_words, tile_words)],
                wsem).wait()
            plsc.subcore_barrier()

    return adj_kernel(edge_flat)


def _edge_gather(edge_flat, idx2e, k):
    mesh = plsc.VectorSubcoreMesh(core_axis_name="c", subcore_axis_name="s")
    n_workers = 32
    per_tile = (2 * k) // n_workers

    @functools.partial(
        pl.kernel,
        out_type=jax.ShapeDtypeStruct((2 * k,), jnp.int32),
        mesh=mesh,
        scratch_types=[
            pltpu.VMEM((per_tile,), jnp.int32),
            pltpu.VMEM((per_tile,), jnp.int32),
            pltpu.SemaphoreType.DMA,
        ],
    )
    def gather_kernel(ef_hbm, i_hbm, out_hbm, iv, ov, sem):
        cid = jax.lax.axis_index("c")
        sid = jax.lax.axis_index("s")
        wid = sid * 2 + cid
        base = wid * per_tile
        pltpu.sync_copy(i_hbm.at[pl.ds(base, per_tile)], iv)
        pltpu.async_copy(ef_hbm.at[iv], ov, sem).wait()
        pltpu.sync_copy(ov, out_hbm.at[pl.ds(base, per_tile)])

    return gather_kernel(edge_flat, idx2e)


def _power_body(gb_ref, rrow_ref, rcol_ref, out_ref, gbq, gbsel, *, blk, mmt):
    n = gbq.shape[0]
    k = gbsel.shape[1]
    i = pl.program_id(0)

    @pl.when(i == 0)
    def _():
        gbq[...] = (gb_ref[...] != 0).astype(mmt)
        qq = jax.lax.broadcasted_iota(jnp.int32, (n, k), 1)
        ptb = (rcol_ref[...] == qq).astype(mmt)
        sel = jax.lax.dot_general(gbq[...], ptb, (((1,), (0,)), ((), ())),
                                  preferred_element_type=jnp.float32)
        gbsel[...] = (sel != 0).astype(mmt)

    rank_row = rrow_ref[...]
    pp = blk * i + jax.lax.broadcasted_iota(jnp.int32, (blk, n), 0)
    p_blk = (rank_row == pp).astype(mmt)

    l1 = jax.lax.dot_general(p_blk, gbq[...], (((1,), (0,)), ((), ())),
                             preferred_element_type=jnp.float32)
    b1 = (l1 != 0).astype(mmt)
    l2 = jax.lax.dot_general(b1, gbq[...], (((1,), (0,)), ((), ())),
                             preferred_element_type=jnp.float32)
    b2 = (l2 != 0).astype(mmt)
    l3 = jax.lax.dot_general(b2, gbsel[...], (((1,), (0,)), ((), ())),
                             preferred_element_type=jnp.float32)
    un = (l3 != 0).astype(jnp.float32)
    rs = jnp.sum(un, axis=1, keepdims=True)
    out_ref[...] = un / (rs + 1e-8)


def _power_norm(gb_flat, rank_row, rank_col, k, interpret=False,
                mmt=jnp.float8_e4m3fn):
    n = rank_row.shape[1]
    blk = 512
    body = functools.partial(_power_body, blk=blk, mmt=mmt)
    return pl.pallas_call(
        body,
        grid=(k // blk,),
        out_shape=jax.ShapeDtypeStruct((k, k), jnp.float32),
        in_specs=[
            pl.BlockSpec((n, n), lambda i: (0, 0)),
            pl.BlockSpec((1, n), lambda i: (0, 0)),
            pl.BlockSpec((n, 1), lambda i: (0, 0)),
        ],
        out_specs=pl.BlockSpec((blk, k), lambda i: (i, 0)),
        scratch_shapes=[
            pltpu.VMEM((n, n), mmt),
            pltpu.VMEM((n, k), mmt),
        ],
        interpret=interpret,
    )(gb_flat, rank_row, rank_col)


def kernel(edge_index, h, C, Wf, bf, Ws, bs, Wo, bo):
    n, d = h.shape
    e = edge_index.shape[1]
    k = max(2, int(RATIO * n))

    feature_weights = h @ Wf + bf
    structure_weights = C @ Ws + bs
    weights = (jnp.concatenate([feature_weights, structure_weights], axis=1)
               @ Wo + bo).squeeze()
    s = jax.nn.sigmoid(weights).reshape(n, 1)

    rank_row, idx, idx2e, new_h = _select(s, s.T, h, k, e)

    edge_flat = edge_index.reshape(2 * e)
    gb_flat = _adj_scatter(edge_flat, n, e)
    new_edge_flat = _edge_gather(edge_flat, idx2e.reshape(2 * k), k)

    g_out = _power_norm(gb_flat.reshape(n, n), rank_row, rank_row.T, k)

    return (g_out, new_h, idx.reshape(k), new_edge_flat.reshape(2, k))
